# baseline (speedup 1.0000x reference)
import jax
import jax.numpy as jnp
from jax import lax
from jax.experimental import pallas as pl
from jax.experimental.pallas import tpu as pltpu
from jax.experimental.pallas import tpu_sc as plsc

BATCH = 16384
EMBED_DIM = 64
NC = 2
NS = 16
NW = NC * NS
BPW = BATCH // NW
CHUNK = 128
NCHUNK = BPW // CHUNK


def _body(uid_hbm, iid_hbm, eu_hbm, ei_hbm, out_hbm,
          idx_u, idx_i, eu_rows, ei_rows, out_v, sem):
    wid = lax.axis_index("s") * NC + lax.axis_index("c")
    base = wid * BPW

    pltpu.sync_copy(uid_hbm.at[pl.ds(wid * NCHUNK, NCHUNK)], idx_u)
    pltpu.sync_copy(iid_hbm.at[pl.ds(wid * NCHUNK, NCHUNK)], idx_i)

    copies = []
    for j in range(NCHUNK):
        copies.append(pltpu.async_copy(
            eu_hbm.at[idx_u.at[j]], eu_rows.at[pl.ds(j * CHUNK, CHUNK)], sem))
        copies.append(pltpu.async_copy(
            ei_hbm.at[idx_i.at[j]], ei_rows.at[pl.ds(j * CHUNK, CHUNK)], sem))
    for c in copies:
        c.wait()

    lanes = lax.iota(jnp.int32, 16)

    def g_body(g, carry):
        row = g * 16 + lanes
        acc = jnp.zeros((16,), jnp.float32)
        for d in range(EMBED_DIM):
            col = jnp.full((16,), d, jnp.int32)
            acc = acc + (plsc.load_gather(eu_rows, [row, col]) *
                         plsc.load_gather(ei_rows, [row, col]))
        out_v[pl.ds(g * 16, 16)] = acc
        return carry

    lax.fori_loop(0, BPW // 16, g_body, 0)

    pltpu.sync_copy(out_v, out_hbm.at[pl.ds(base, BPW)])


@jax.jit
def kernel(x, embed_user, embed_item):
    uid = x[:, 0].astype(jnp.int32).reshape(NW * NCHUNK, CHUNK)
    iid = x[:, 1].astype(jnp.int32).reshape(NW * NCHUNK, CHUNK)

    run = pl.kernel(
        _body,
        out_type=jax.ShapeDtypeStruct((BATCH,), jnp.float32),
        mesh=plsc.VectorSubcoreMesh(core_axis_name="c", subcore_axis_name="s"),
        compiler_params=pltpu.CompilerParams(
            needs_layout_passes=False, use_tc_tiling_on_sc=False),
        scratch_types=[
            pltpu.VMEM((NCHUNK, CHUNK), jnp.int32),
            pltpu.VMEM((NCHUNK, CHUNK), jnp.int32),
            pltpu.VMEM((BPW, EMBED_DIM), jnp.float32),
            pltpu.VMEM((BPW, EMBED_DIM), jnp.float32),
            pltpu.VMEM((BPW,), jnp.float32),
            pltpu.SemaphoreType.DMA,
        ],
    )
    return run(uid, iid, embed_user, embed_item)


# speedup vs baseline: 1.5358x; 1.5358x over previous
import jax
import jax.numpy as jnp
from jax import lax
from jax.experimental import pallas as pl
from jax.experimental.pallas import tpu as pltpu
from jax.experimental.pallas import tpu_sc as plsc

BATCH = 16384
EMBED_DIM = 64
NC = 2
NS = 16
NW = NC * NS
BPW = BATCH // NW
GROUPS = BPW // 16


def _body(uid_hbm, iid_hbm, eu_hbm, ei_hbm, out_hbm,
          idx_u, idx_i, eu_rows, ei_rows, out_v, sem):
    wid = lax.axis_index("s") * NC + lax.axis_index("c")
    base = wid * BPW

    pltpu.sync_copy(uid_hbm.at[pl.ds(wid * 8, 8)], idx_u)
    pltpu.sync_copy(iid_hbm.at[pl.ds(wid * 8, 8)], idx_i)

    lanes = lax.iota(jnp.int32, 16)

    def step(j, carry):
        copies = []
        for k in range(4):
            uvec = idx_u[j, pl.ds(k * 16, 16)]
            ivec = idx_i[j, pl.ds(k * 16, 16)]
            for l in range(16):
                p = k * 16 + l
                copies.append(pltpu.async_copy(
                    eu_hbm.at[uvec[l]], eu_rows.at[p], sem))
                copies.append(pltpu.async_copy(
                    ei_hbm.at[ivec[l]], ei_rows.at[p], sem))
        for c in copies:
            c.wait()
        for g in range(4):
            rowvec = g * 16 + lanes
            acc = jnp.zeros((16,), jnp.float32)
            for d in range(EMBED_DIM):
                col = jnp.full((16,), d, jnp.int32)
                acc = acc + (plsc.load_gather(eu_rows, [rowvec, col]) *
                             plsc.load_gather(ei_rows, [rowvec, col]))
            out_v[pl.ds(j * 64 + g * 16, 16)] = acc
        return carry

    lax.fori_loop(0, 8, step, 0)

    pltpu.sync_copy(out_v, out_hbm.at[pl.ds(base, BPW)])


@jax.jit
def kernel(x, embed_user, embed_item):
    uid = x[:, 0].astype(jnp.int32).reshape(NW * 8, 64)
    iid = x[:, 1].astype(jnp.int32).reshape(NW * 8, 64)

    run = pl.kernel(
        _body,
        out_type=jax.ShapeDtypeStruct((BATCH,), jnp.float32),
        mesh=plsc.VectorSubcoreMesh(core_axis_name="c", subcore_axis_name="s"),
        compiler_params=pltpu.CompilerParams(needs_layout_passes=False),
        scratch_types=[
            pltpu.VMEM((8, 64), jnp.int32),
            pltpu.VMEM((8, 64), jnp.int32),
            pltpu.VMEM((64, EMBED_DIM), jnp.float32),
            pltpu.VMEM((64, EMBED_DIM), jnp.float32),
            pltpu.VMEM((BPW,), jnp.float32),
            pltpu.SemaphoreType.DMA,
        ],
    )
    return run(uid, iid, embed_user, embed_item)


# speedup vs baseline: 4.2873x; 2.7915x over previous
import jax
import jax.numpy as jnp
from jax import lax
from jax.experimental import pallas as pl
from jax.experimental.pallas import tpu as pltpu
from jax.experimental.pallas import tpu_sc as plsc

BATCH = 16384
EMBED_DIM = 64
NROWS = 1000000
NC = 2
NS = 16
NW = NC * NS
BPW = BATCH // NW
CCOLS = 512
CPW = 61
WCOLS = CPW * CCOLS
TAIL0 = 1953 * CCOLS
WLCAP = 4096


def _sweep_body(uid_hbm, iid_hbm, euT_hbm, eiT_hbm, ru_hbm, ri_hbm,
                ids_v, wl_id, wl_pos, cl_id, cl_pos, chbuf, tailbuf,
                rowtmp, csem, rsem):
    wid = lax.axis_index("s") * NC + lax.axis_index("c")
    lanes = lax.iota(jnp.int32, 16)
    lo = wid * WCOLS
    hi = jnp.where(wid == NW - 1, NROWS, lo + WCOLS)
    nch = jnp.where(wid == NW - 1, CPW + 1, CPW)

    def one_table(ids_hbm, tab_hbm, rows_hbm):
        pltpu.sync_copy(ids_hbm, ids_v)

        def scan(g, cnt):
            v = ids_v[pl.ds(g * 16, 16)]
            pos = g * 16 + lanes
            m = jnp.logical_and(v >= lo, v < hi)
            plsc.store_compressed(wl_id.at[pl.ds(cnt, 16)], v, mask=m)
            plsc.store_compressed(wl_pos.at[pl.ds(cnt, 16)], pos, mask=m)
            pc = plsc.all_reduce_population_count(m)
            return jnp.minimum(cnt + pc[0], WLCAP)

        cnt = lax.fori_loop(0, BATCH // 16, scan, jnp.int32(0))
        nk = (cnt + 15) // 16

        def extract_group(e, ccnt, buf, gather_fn):
            rem = ccnt - e * 16
            lc = cl_id[pl.ds(e * 16, 16)]
            pp = cl_pos[pl.ds(e * 16, 16)]
            for l in range(16):

                @pl.when(l < rem)
                def _():
                    lcv = jnp.broadcast_to(lc[l], (16,))
                    for k in range(4):
                        dvec = k * 16 + lanes
                        rowtmp[l, pl.ds(k * 16, 16)] = gather_fn(dvec, lcv)
                    pltpu.async_copy(rowtmp.at[l], rows_hbm.at[pp[l]], rsem)

            for l in range(16):

                @pl.when(l < rem)
                def _():
                    pltpu.make_async_copy(
                        rowtmp.at[l], rows_hbm.at[0], rsem).wait()
            return ccnt

        def chunk_rescan(base, width):
            def rescan(k, ccnt):
                wv = wl_id[pl.ds(k * 16, 16)]
                wp = wl_pos[pl.ds(k * 16, 16)]
                m = jnp.logical_and(
                    jnp.logical_and(wv >= base, wv < base + width),
                    k * 16 + lanes < cnt)
                plsc.store_compressed(
                    cl_id.at[pl.ds(ccnt, 16)], wv - base, mask=m)
                plsc.store_compressed(
                    cl_pos.at[pl.ds(ccnt, 16)], wp, mask=m)
                pc = plsc.all_reduce_population_count(m)
                return ccnt + pc[0]

            return lax.fori_loop(0, nk, rescan, jnp.int32(0))

        pltpu.async_copy(
            tab_hbm.at[:, pl.ds(lo, CCOLS)], chbuf.at[0], csem)

        def chunk(c, carry):
            par = lax.rem(c, 2)
            base = lo + c * CCOLS

            @pl.when(c + 1 < nch)
            def _():
                pltpu.async_copy(
                    tab_hbm.at[:, pl.ds(base + CCOLS, CCOLS)],
                    chbuf.at[lax.rem(c + 1, 2)], csem)

            pltpu.make_async_copy(
                tab_hbm.at[:, pl.ds(0, CCOLS)], chbuf.at[par], csem).wait()

            ccnt = chunk_rescan(base, CCOLS)
            parv = jnp.broadcast_to(par, (16,))

            def gather_fn(dvec, lcv):
                return plsc.load_gather(chbuf, [parv, dvec, lcv])

            lax.fori_loop(
                0, (ccnt + 15) // 16,
                lambda e, a: extract_group(e, ccnt, chbuf, gather_fn), ccnt)
            return carry

        lax.fori_loop(0, nch, chunk, 0)

        @pl.when(wid == NW - 1)
        def _():
            pltpu.sync_copy(tab_hbm.at[:, pl.ds(TAIL0, NROWS - TAIL0)],
                            tailbuf)
            ccnt = chunk_rescan(TAIL0, NROWS - TAIL0)

            def gather_fn(dvec, lcv):
                return plsc.load_gather(tailbuf, [dvec, lcv])

            lax.fori_loop(
                0, (ccnt + 15) // 16,
                lambda e, a: extract_group(e, ccnt, tailbuf, gather_fn), ccnt)

    one_table(uid_hbm, euT_hbm, ru_hbm)
    one_table(iid_hbm, eiT_hbm, ri_hbm)


def _dot_body(ru_hbm, ri_hbm, out_hbm, bu, bi, ov):
    wid = lax.axis_index("s") * NC + lax.axis_index("c")
    lanes = lax.iota(jnp.int32, 16)
    pltpu.sync_copy(ru_hbm.at[wid], bu)
    pltpu.sync_copy(ri_hbm.at[wid], bi)

    def group(g, carry):
        bvec = (g * 16 + lanes) * EMBED_DIM
        acc = jnp.zeros((16,), jnp.float32)
        for d0 in range(EMBED_DIM):
            idx = bvec + jnp.bitwise_and(d0 + lanes, EMBED_DIM - 1)
            acc = acc + plsc.load_gather(bu, [idx]) * plsc.load_gather(
                bi, [idx])
        ov[pl.ds(g * 16, 16)] = acc
        return carry

    lax.fori_loop(0, BPW // 16, group, 0)
    pltpu.sync_copy(ov, out_hbm.at[pl.ds(wid * BPW, BPW)])


@jax.jit
def kernel(x, embed_user, embed_item):
    uid = x[:, 0].astype(jnp.int32)
    iid = x[:, 1].astype(jnp.int32)
    euT = embed_user.T
    eiT = embed_item.T

    mesh = plsc.VectorSubcoreMesh(core_axis_name="c", subcore_axis_name="s")
    params = pltpu.CompilerParams(needs_layout_passes=False)

    sweep = pl.kernel(
        _sweep_body,
        out_type=(
            jax.ShapeDtypeStruct((BATCH, EMBED_DIM), jnp.float32),
            jax.ShapeDtypeStruct((BATCH, EMBED_DIM), jnp.float32),
        ),
        mesh=mesh,
        compiler_params=params,
        scratch_types=[
            pltpu.VMEM((BATCH,), jnp.int32),
            pltpu.VMEM((WLCAP + 16,), jnp.int32),
            pltpu.VMEM((WLCAP + 16,), jnp.int32),
            pltpu.VMEM((WLCAP + 16,), jnp.int32),
            pltpu.VMEM((WLCAP + 16,), jnp.int32),
            pltpu.VMEM((2, EMBED_DIM, CCOLS), jnp.float32),
            pltpu.VMEM((EMBED_DIM, NROWS - TAIL0), jnp.float32),
            pltpu.VMEM((16, EMBED_DIM), jnp.float32),
            pltpu.SemaphoreType.DMA,
            pltpu.SemaphoreType.DMA,
        ],
    )
    ru, ri = sweep(uid, iid, euT, eiT)

    dot = pl.kernel(
        _dot_body,
        out_type=jax.ShapeDtypeStruct((BATCH,), jnp.float32),
        mesh=mesh,
        compiler_params=params,
        scratch_types=[
            pltpu.VMEM((BPW * EMBED_DIM,), jnp.float32),
            pltpu.VMEM((BPW * EMBED_DIM,), jnp.float32),
            pltpu.VMEM((BPW,), jnp.float32),
        ],
    )
    return dot(ru.reshape(NW, BPW * EMBED_DIM), ri.reshape(NW, BPW * EMBED_DIM))


# speedup vs baseline: 4.6761x; 1.0907x over previous
import jax
import jax.numpy as jnp
from jax import lax
from jax.experimental import pallas as pl
from jax.experimental.pallas import tpu as pltpu
from jax.experimental.pallas import tpu_sc as plsc

BATCH = 16384
EMBED_DIM = 64
NROWS = 1000000
NC = 2
NS = 16
NW = NC * NS
BPW = BATCH // NW
CCOLS = 512
CPW = 61
WCOLS = CPW * CCOLS
TAIL0 = 1953 * CCOLS
WLCAP = 1024
NBUF = 3


def _sweep_body(uid_hbm, iid_hbm, euT_hbm, eiT_hbm, ru_hbm, ri_hbm,
                ids_v, wl_id, wl_pos, cl_id, cl_pos, chbuf, tailbuf,
                rowtmp, csem, rsem):
    wid = lax.axis_index("s") * NC + lax.axis_index("c")
    lanes = lax.iota(jnp.int32, 16)
    lo = wid * WCOLS
    hi = jnp.where(wid == NW - 1, NROWS, lo + WCOLS)
    nch = jnp.where(wid == NW - 1, CPW + 1, CPW)

    def one_table(ids_hbm, tab_hbm, rows_hbm):
        pltpu.sync_copy(ids_hbm, ids_v)

        def scan(g, cnt):
            v = ids_v[pl.ds(g * 16, 16)]
            pos = g * 16 + lanes
            m = jnp.logical_and(v >= lo, v < hi)
            plsc.store_compressed(wl_id.at[pl.ds(cnt, 16)], v, mask=m)
            plsc.store_compressed(wl_pos.at[pl.ds(cnt, 16)], pos, mask=m)
            pc = plsc.all_reduce_population_count(m)
            return jnp.minimum(cnt + pc[0], WLCAP)

        cnt = lax.fori_loop(0, BATCH // 16, scan, jnp.int32(0))
        nk = (cnt + 15) // 16

        def extract_group(e, ccnt, buf, gather_fn):
            rem = ccnt - e * 16
            lc = cl_id[pl.ds(e * 16, 16)]
            pp = cl_pos[pl.ds(e * 16, 16)]
            for l in range(16):

                @pl.when(l < rem)
                def _():
                    lcv = jnp.broadcast_to(lc[l], (16,))
                    for k in range(4):
                        dvec = k * 16 + lanes
                        rowtmp[l, pl.ds(k * 16, 16)] = gather_fn(dvec, lcv)
                    pltpu.async_copy(rowtmp.at[l], rows_hbm.at[pp[l]], rsem)

            for l in range(16):

                @pl.when(l < rem)
                def _():
                    pltpu.make_async_copy(
                        rowtmp.at[l], rows_hbm.at[0], rsem).wait()
            return ccnt

        def chunk_rescan(base, width):
            def rescan(k, ccnt):
                wv = wl_id[pl.ds(k * 16, 16)]
                wp = wl_pos[pl.ds(k * 16, 16)]
                m = jnp.logical_and(
                    jnp.logical_and(wv >= base, wv < base + width),
                    k * 16 + lanes < cnt)
                plsc.store_compressed(
                    cl_id.at[pl.ds(ccnt, 16)], wv - base, mask=m)
                plsc.store_compressed(
                    cl_pos.at[pl.ds(ccnt, 16)], wp, mask=m)
                pc = plsc.all_reduce_population_count(m)
                return ccnt + pc[0]

            return lax.fori_loop(0, nk, rescan, jnp.int32(0))

        for pre in range(NBUF - 1):

            @pl.when(pre < nch)
            def _():
                pltpu.async_copy(
                    tab_hbm.at[:, pl.ds(lo + pre * CCOLS, CCOLS)],
                    chbuf.at[pre], csem)

        def chunk(c, carry):
            par = lax.rem(c, NBUF)
            base = lo + c * CCOLS

            @pl.when(c + NBUF - 1 < nch)
            def _():
                pltpu.async_copy(
                    tab_hbm.at[:, pl.ds(base + (NBUF - 1) * CCOLS, CCOLS)],
                    chbuf.at[lax.rem(c + NBUF - 1, NBUF)], csem)

            pltpu.make_async_copy(
                tab_hbm.at[:, pl.ds(0, CCOLS)], chbuf.at[par], csem).wait()

            ccnt = chunk_rescan(base, CCOLS)
            parv = jnp.broadcast_to(par, (16,))

            def gather_fn(dvec, lcv):
                return plsc.load_gather(chbuf, [parv, dvec, lcv])

            lax.fori_loop(
                0, (ccnt + 15) // 16,
                lambda e, a: extract_group(e, ccnt, chbuf, gather_fn), ccnt)
            return carry

        lax.fori_loop(0, nch, chunk, 0)

        @pl.when(wid == NW - 1)
        def _():
            pltpu.sync_copy(tab_hbm.at[:, pl.ds(TAIL0, NROWS - TAIL0)],
                            tailbuf)
            ccnt = chunk_rescan(TAIL0, NROWS - TAIL0)

            def gather_fn(dvec, lcv):
                return plsc.load_gather(tailbuf, [dvec, lcv])

            lax.fori_loop(
                0, (ccnt + 15) // 16,
                lambda e, a: extract_group(e, ccnt, tailbuf, gather_fn), ccnt)

    one_table(uid_hbm, euT_hbm, ru_hbm)
    one_table(iid_hbm, eiT_hbm, ri_hbm)


def _dot_body(ru_hbm, ri_hbm, out_hbm, bu, bi, ov):
    wid = lax.axis_index("s") * NC + lax.axis_index("c")
    lanes = lax.iota(jnp.int32, 16)
    pltpu.sync_copy(ru_hbm.at[wid], bu)
    pltpu.sync_copy(ri_hbm.at[wid], bi)

    def group(g, carry):
        bvec = (g * 16 + lanes) * EMBED_DIM
        acc = jnp.zeros((16,), jnp.float32)
        for d0 in range(EMBED_DIM):
            idx = bvec + jnp.bitwise_and(d0 + lanes, EMBED_DIM - 1)
            acc = acc + plsc.load_gather(bu, [idx]) * plsc.load_gather(
                bi, [idx])
        ov[pl.ds(g * 16, 16)] = acc
        return carry

    lax.fori_loop(0, BPW // 16, group, 0)
    pltpu.sync_copy(ov, out_hbm.at[pl.ds(wid * BPW, BPW)])


@jax.jit
def kernel(x, embed_user, embed_item):
    uid = x[:, 0].astype(jnp.int32)
    iid = x[:, 1].astype(jnp.int32)
    euT = embed_user.T
    eiT = embed_item.T

    mesh = plsc.VectorSubcoreMesh(core_axis_name="c", subcore_axis_name="s")
    params = pltpu.CompilerParams(needs_layout_passes=False)

    sweep = pl.kernel(
        _sweep_body,
        out_type=(
            jax.ShapeDtypeStruct((BATCH, EMBED_DIM), jnp.float32),
            jax.ShapeDtypeStruct((BATCH, EMBED_DIM), jnp.float32),
        ),
        mesh=mesh,
        compiler_params=params,
        scratch_types=[
            pltpu.VMEM((BATCH,), jnp.int32),
            pltpu.VMEM((WLCAP + 16,), jnp.int32),
            pltpu.VMEM((WLCAP + 16,), jnp.int32),
            pltpu.VMEM((WLCAP + 16,), jnp.int32),
            pltpu.VMEM((WLCAP + 16,), jnp.int32),
            pltpu.VMEM((NBUF, EMBED_DIM, CCOLS), jnp.float32),
            pltpu.VMEM((EMBED_DIM, NROWS - TAIL0), jnp.float32),
            pltpu.VMEM((16, EMBED_DIM), jnp.float32),
            pltpu.SemaphoreType.DMA,
            pltpu.SemaphoreType.DMA,
        ],
    )
    ru, ri = sweep(uid, iid, euT, eiT)

    dot = pl.kernel(
        _dot_body,
        out_type=jax.ShapeDtypeStruct((BATCH,), jnp.float32),
        mesh=mesh,
        compiler_params=params,
        scratch_types=[
            pltpu.VMEM((BPW * EMBED_DIM,), jnp.float32),
            pltpu.VMEM((BPW * EMBED_DIM,), jnp.float32),
            pltpu.VMEM((BPW,), jnp.float32),
        ],
    )
    return dot(ru.reshape(NW, BPW * EMBED_DIM), ri.reshape(NW, BPW * EMBED_DIM))
